# Initial kernel scaffold; baseline (speedup 1.0000x reference)
import jax
import jax.numpy as jnp
from jax.experimental import pallas as pl


def kernel(inputs_BL, emb, W1, b1, W2, b2, W3, b3):
    raise NotImplementedError("write your pallas kernel here")


# speedup vs baseline: 1.2456x; 1.2456x over previous
import functools

import jax
import jax.numpy as jnp
from jax import lax
from jax.experimental import pallas as pl
from jax.experimental.pallas import tpu as pltpu
from jax.experimental.pallas import tpu_sc as plsc

B, L, V, E, H, C = 16384, 5, 1000000, 64, 128, 2

_NC, _NS = 2, 16
_NW = _NC * _NS
_N = B * L
_PW = _N // _NW
_CH = 128
_NCH = _PW // _CH
_NBUF = 4

_sc_mesh = plsc.VectorSubcoreMesh(core_axis_name="c", subcore_axis_name="s")


@functools.partial(
    pl.kernel,
    out_type=jax.ShapeDtypeStruct((_N, E), jnp.float32),
    mesh=_sc_mesh,
    scratch_types=[
        pltpu.VMEM((_NCH, _CH), jnp.int32),
        pltpu.VMEM((_NBUF, _CH, E), jnp.float32),
    ] + [pltpu.SemaphoreType.DMA] * _NBUF,
)
def _sc_gather(idx_hbm, emb_hbm, out_hbm, idx_v, rows_v, *sems):
    wid = lax.axis_index("s") * _NC + lax.axis_index("c")
    base = wid * _PW
    pltpu.sync_copy(idx_hbm.at[wid], idx_v)

    def start(i):
        bi = i % _NBUF
        return pltpu.async_copy(emb_hbm.at[idx_v.at[i]], rows_v.at[bi], sems[bi])

    handles = {}
    for i in range(min(_NBUF, _NCH)):
        handles[i] = start(i)
    for i in range(_NCH):
        bi = i % _NBUF
        handles.pop(i).wait()
        pltpu.sync_copy(rows_v.at[bi], out_hbm.at[pl.ds(base + i * _CH, _CH)])
        j = i + _NBUF
        if j < _NCH:
            handles[j] = start(j)


def _mlp_body(x_ref, w1_ref, b1_ref, w2_ref, b2_ref, w3_ref, b3_ref, o_ref):
    x = x_ref[...]
    h = jnp.maximum(
        jnp.dot(x, w1_ref[...], preferred_element_type=jnp.float32) + b1_ref[...], 0.0)
    h = jnp.maximum(
        jnp.dot(h, w2_ref[...], preferred_element_type=jnp.float32) + b2_ref[...], 0.0)
    o = jnp.dot(h, w3_ref[...], preferred_element_type=jnp.float32) + b3_ref[...]
    col = lax.broadcasted_iota(jnp.int32, o.shape, 1)
    o = jnp.where(col < C, o, jnp.float32(-1e30))
    m = jnp.max(o, axis=1, keepdims=True)
    e = jnp.exp(o - m)
    o_ref[...] = e / jnp.sum(e, axis=1, keepdims=True)


_BLK = 2048

_mlp = pl.pallas_call(
    _mlp_body,
    grid=(B // _BLK,),
    in_specs=[
        pl.BlockSpec((_BLK, L * E), lambda i: (i, 0)),
        pl.BlockSpec((L * E, H), lambda i: (0, 0)),
        pl.BlockSpec((1, H), lambda i: (0, 0)),
        pl.BlockSpec((H, H), lambda i: (0, 0)),
        pl.BlockSpec((1, H), lambda i: (0, 0)),
        pl.BlockSpec((H, H), lambda i: (0, 0)),
        pl.BlockSpec((1, H), lambda i: (0, 0)),
    ],
    out_specs=pl.BlockSpec((_BLK, H), lambda i: (i, 0)),
    out_shape=jax.ShapeDtypeStruct((B, H), jnp.float32),
)


def kernel(inputs_BL, emb, W1, b1, W2, b2, W3, b3):
    idx = inputs_BL.astype(jnp.int32).reshape(-1)
    rows = jnp.take(emb, idx, axis=0)
    x = rows.reshape(B, L * E)
    w3p = jnp.pad(W3, ((0, 0), (0, H - C)))
    b3p = jnp.pad(b3, (0, H - C))
    out = _mlp(x, W1, b1.reshape(1, H), W2, b2.reshape(1, H), w3p, b3p.reshape(1, H))
    return out[:, :C]


# speedup vs baseline: 1.3614x; 1.0930x over previous
import jax
import jax.numpy as jnp
from jax import lax
from jax.experimental import pallas as pl

B, L, V, E, H, C = 16384, 5, 1000000, 64, 128, 2

_BLK = 2048


def _mlp_t_body(x_refs, w1t_ref, b1_ref, w2t_ref, b2_ref, w3t_ref, b3_ref, o_ref):
    w1t = w1t_ref[...]
    acc = jnp.zeros((H, _BLK), jnp.float32)
    for l in range(L):
        acc = acc + jnp.dot(w1t[:, l * E:(l + 1) * E], x_refs[l][...],
                            preferred_element_type=jnp.float32)
    h = jnp.maximum(acc + b1_ref[...], 0.0)
    h = jnp.maximum(jnp.dot(w2t_ref[...], h,
                            preferred_element_type=jnp.float32) + b2_ref[...], 0.0)
    o = jnp.dot(w3t_ref[...], h, preferred_element_type=jnp.float32) + b3_ref[...]
    row = lax.broadcasted_iota(jnp.int32, o.shape, 0)
    o = jnp.where(row < C, o, jnp.float32(-1e30))
    m = jnp.max(o, axis=0, keepdims=True)
    e = jnp.exp(o - m)
    o_ref[...] = e / jnp.sum(e, axis=0, keepdims=True)


def _mlp_t_entry(x0, x1, x2, x3, x4, w1t, b1, w2t, b2, w3t, b3, o):
    _mlp_t_body((x0, x1, x2, x3, x4), w1t, b1, w2t, b2, w3t, b3, o)


_NB = B // _BLK


def _x_spec(l):
    return pl.BlockSpec((E, _BLK), lambda i, _l=l: (0, _l * _NB + i))


_mlp_t = pl.pallas_call(
    _mlp_t_entry,
    grid=(_NB,),
    in_specs=[_x_spec(l) for l in range(L)] + [
        pl.BlockSpec((H, L * E), lambda i: (0, 0)),
        pl.BlockSpec((H, 1), lambda i: (0, 0)),
        pl.BlockSpec((H, H), lambda i: (0, 0)),
        pl.BlockSpec((H, 1), lambda i: (0, 0)),
        pl.BlockSpec((H, H), lambda i: (0, 0)),
        pl.BlockSpec((H, 1), lambda i: (0, 0)),
    ],
    out_specs=pl.BlockSpec((H, _BLK), lambda i: (0, i)),
    out_shape=jax.ShapeDtypeStruct((H, B), jnp.float32),
)


def kernel(inputs_BL, emb, W1, b1, W2, b2, W3, b3):
    idx = inputs_BL.astype(jnp.int32).T.reshape(-1)
    rows = jnp.take(emb, idx, axis=0)
    xT = rows.T
    w3p = jnp.pad(W3, ((0, 0), (0, H - C)))
    oT = _mlp_t(
        xT, xT, xT, xT, xT,
        W1.T, b1.reshape(H, 1),
        W2.T, b2.reshape(H, 1),
        w3p.T, jnp.pad(b3, (0, H - C)).reshape(H, 1),
    )
    return oT[:C, :].T
